# baseline (speedup 1.0000x reference)
import functools

import jax
import jax.numpy as jnp
from jax.experimental import pallas as pl
from jax.experimental.pallas import tpu as pltpu

_LANES = 128


def _round_up(a: int, m: int) -> int:
    return (a + m - 1) // m * m


def _cdiv(a: int, m: int) -> int:
    return (a + m - 1) // m


def _fused_body(x_ref, out_ref, *, inv_len):
    x = x_ref[...].astype(jnp.float32)
    out_ref[:, 0] = jnp.max(x, axis=2, keepdims=True).astype(out_ref.dtype)
    out_ref[:, 1] = (jnp.sum(x, axis=2, keepdims=True)
                     * inv_len).astype(out_ref.dtype)


def _pool_body(x_ref, max_ref, avg_ref, *, inv_len):
    x = x_ref[...].astype(jnp.float32)
    max_ref[...] = jnp.max(x, axis=1, keepdims=True).astype(max_ref.dtype)
    avg_ref[...] = (jnp.sum(x, axis=1, keepdims=True)
                    * inv_len).astype(avg_ref.dtype)


def _concat_pool_fused(x, *, batches_per_block):
    N, C, L = x.shape
    nb = batches_per_block
    body = functools.partial(_fused_body, inv_len=1.0 / L)
    out = pl.pallas_call(
        body,
        out_shape=jax.ShapeDtypeStruct((N, 2, C, 1), x.dtype),
        grid=(_cdiv(N, nb),),
        in_specs=[pl.BlockSpec((nb, C, L), lambda i: (i, 0, 0))],
        out_specs=pl.BlockSpec((nb, 2, C, 1), lambda i: (i, 0, 0, 0)),
        compiler_params=pltpu.CompilerParams(
            dimension_semantics=("parallel",)),
    )(x)
    return out.reshape(N, 2 * C, 1)


def _concat_pool_rows(x, *, target_block_bytes=8 * 1024 * 1024):
    N, C, L = x.shape
    NR = N * C
    x2 = x.reshape(NR, L)

    sub = {4: 8, 2: 16, 1: 32}.get(jnp.dtype(x.dtype).itemsize, 8)
    row_bytes = L * jnp.dtype(x.dtype).itemsize
    br = max(sub, _round_up(max(1, target_block_bytes // row_bytes), sub))
    if NR > sub:
        br = min(br, _round_up(_cdiv(NR, 2), sub))
    br = min(br, _round_up(NR, sub))
    nr_blocks = _cdiv(NR, br)

    body = functools.partial(_pool_body, inv_len=1.0 / L)
    mx2, av2 = pl.pallas_call(
        body,
        out_shape=(jax.ShapeDtypeStruct((NR, 1), x.dtype),
                   jax.ShapeDtypeStruct((NR, 1), x.dtype)),
        grid=(nr_blocks,),
        in_specs=[pl.BlockSpec((br, L), lambda i: (i, 0))],
        out_specs=[pl.BlockSpec((br, 1), lambda i: (i, 0)),
                   pl.BlockSpec((br, 1), lambda i: (i, 0))],
        compiler_params=pltpu.CompilerParams(
            dimension_semantics=("parallel",)),
    )(x2)

    mx = mx2.reshape(N, C)
    av = av2.reshape(N, C)
    return jnp.concatenate([mx, av], axis=1)[:, :, None]


def _dual_body(a_ref, b_ref, mx_ref, av_ref, *, inv_len):
    a = a_ref[0].astype(jnp.float32)
    b = b_ref[0].astype(jnp.float32)
    mx_ref[0] = jnp.max(a, axis=1, keepdims=True).astype(mx_ref.dtype)
    mx_ref[1] = jnp.max(b, axis=1, keepdims=True).astype(mx_ref.dtype)
    av_ref[0] = (jnp.sum(a, axis=1, keepdims=True)
                 * inv_len).astype(av_ref.dtype)
    av_ref[1] = (jnp.sum(b, axis=1, keepdims=True)
                 * inv_len).astype(av_ref.dtype)


def _concat_pool_dual(x, *, br):
    N, C, L = x.shape
    NR = N * C
    NR2 = NR // 2
    x3 = x.reshape(2, NR2, L)
    body = functools.partial(_dual_body, inv_len=1.0 / L)
    mx3, av3 = pl.pallas_call(
        body,
        out_shape=(jax.ShapeDtypeStruct((2, NR2, 1), x.dtype),
                   jax.ShapeDtypeStruct((2, NR2, 1), x.dtype)),
        grid=(_cdiv(NR2, br),),
        in_specs=[pl.BlockSpec((1, br, L), lambda i: (0, i, 0)),
                  pl.BlockSpec((1, br, L), lambda i: (1, i, 0))],
        out_specs=[pl.BlockSpec((2, br, 1), lambda i: (0, i, 0)),
                   pl.BlockSpec((2, br, 1), lambda i: (0, i, 0))],
        compiler_params=pltpu.CompilerParams(
            dimension_semantics=("parallel",)),
    )(x3, x3)

    mx = mx3.reshape(N, C)
    av = av3.reshape(N, C)
    return jnp.concatenate([mx, av], axis=1)[:, :, None]


def kernel(x):
    N, C, L = x.shape
    block_bytes = C * L * jnp.dtype(x.dtype).itemsize
    if False and C % 8 == 0 and N >= 2 and block_bytes <= 16 * 1024 * 1024:
        nb = max(1, (20 * 1024 * 1024) // block_bytes)
        while nb > 1 and _cdiv(N, nb) < 2:
            nb //= 2
        return _concat_pool_fused(x, batches_per_block=nb)
    NR = N * C
    if NR % 16 == 0 and (NR // 2) % 8 == 0:
        return _concat_pool_dual(x, br=256)
    return _concat_pool_rows(x, target_block_bytes=8 * 1024 * 1024)


# speedup vs baseline: 1.0180x; 1.0180x over previous
import functools

import jax
import jax.numpy as jnp
from jax.experimental import pallas as pl
from jax.experimental.pallas import tpu as pltpu

_LANES = 128
_TARGET_BLOCK_BYTES = 8 * 1024 * 1024
_MAX_SINGLE_REDUCE_L = 32768


def _round_up(a: int, m: int) -> int:
    return (a + m - 1) // m * m


def _cdiv(a: int, m: int) -> int:
    return (a + m - 1) // m


def _pool_body(x_ref, max_ref, avg_ref, *, inv_len):
    x = x_ref[...].astype(jnp.float32)
    max_ref[...] = jnp.max(x, axis=1, keepdims=True).astype(max_ref.dtype)
    avg_ref[...] = (jnp.sum(x, axis=1, keepdims=True)
                    * inv_len).astype(avg_ref.dtype)


def _pool_body_chunked(x_ref, max_ref, avg_ref, *, n_chunks, length, inv_len):
    acc_m = x_ref[:, :_LANES].astype(jnp.float32)
    acc_s = acc_m
    for q in range(1, n_chunks):
        lo = q * _LANES
        xq = x_ref[:, lo:lo + _LANES].astype(jnp.float32)
        if lo + _LANES > length:
            col = lo + jax.lax.broadcasted_iota(jnp.int32, xq.shape, 1)
            valid = col < length
            acc_m = jnp.maximum(acc_m, jnp.where(valid, xq, -jnp.inf))
            acc_s = acc_s + jnp.where(valid, xq, 0.0)
        else:
            acc_m = jnp.maximum(acc_m, xq)
            acc_s = acc_s + xq
    max_ref[...] = jnp.max(acc_m, axis=1, keepdims=True).astype(max_ref.dtype)
    avg_ref[...] = (jnp.sum(acc_s, axis=1, keepdims=True)
                    * inv_len).astype(avg_ref.dtype)


def kernel(x):
    N, C, L = x.shape
    NR = N * C
    x2 = x.reshape(NR, L)

    sub = {4: 8, 2: 16, 1: 32}.get(jnp.dtype(x.dtype).itemsize, 8)
    row_bytes = L * jnp.dtype(x.dtype).itemsize
    br = max(sub, _round_up(max(1, _TARGET_BLOCK_BYTES // row_bytes), sub))
    if NR > sub:
        br = min(br, _round_up(_cdiv(NR, 2), sub))
    br = min(br, _round_up(NR, sub))
    nr_blocks = _cdiv(NR, br)

    if L <= _MAX_SINGLE_REDUCE_L:
        body = functools.partial(_pool_body, inv_len=1.0 / L)
    else:
        body = functools.partial(_pool_body_chunked,
                                 n_chunks=_cdiv(L, _LANES), length=L,
                                 inv_len=1.0 / L)

    mx2, av2 = pl.pallas_call(
        body,
        out_shape=(jax.ShapeDtypeStruct((NR, 1), x.dtype),
                   jax.ShapeDtypeStruct((NR, 1), x.dtype)),
        grid=(nr_blocks,),
        in_specs=[pl.BlockSpec((br, L), lambda i: (i, 0))],
        out_specs=[pl.BlockSpec((br, 1), lambda i: (i, 0)),
                   pl.BlockSpec((br, 1), lambda i: (i, 0))],
        compiler_params=pltpu.CompilerParams(
            dimension_semantics=("parallel",)),
    )(x2)

    mx = mx2.reshape(N, C)
    av = av2.reshape(N, C)
    return jnp.concatenate([mx, av], axis=1)[:, :, None]
